# baseline (speedup 1.0000x reference)
import functools

import jax
import jax.numpy as jnp
import numpy as np
from jax import lax
from jax.experimental import pallas as pl
from jax.experimental.pallas import tpu as pltpu
from jax.experimental.pallas import tpu_sc as plsc

N_PLAYER = 1000000
S = 8
B = 16384
T = 20

NC = 2
NS = 16
NW = NC * NS
BH = B
TPW = (BH * T) // NW

_K1 = np.uint32(0)
_K2 = np.uint32(42)

_mesh = plsc.VectorSubcoreMesh(core_axis_name="c", subcore_axis_name="s")


@functools.partial(
    pl.kernel,
    out_type=(
        jax.ShapeDtypeStruct((BH * T,), jnp.float32),
        jax.ShapeDtypeStruct((BH * T,), jnp.float32),
    ),
    mesh=_mesh,
    scratch_types=[
        pltpu.VMEM((TPW,), jnp.int32),
        pltpu.VMEM((TPW,), jnp.float32),
        pltpu.VMEM((TPW,), jnp.float32),
        pltpu.SemaphoreType.DMA,
        pltpu.SemaphoreType.DMA,
    ],
    compiler_params=pltpu.CompilerParams(use_tc_tiling_on_sc=False),
)
def _gather_sc(teamt_hbm, mu_hbm, rho_hbm, mug_hbm, rhog_hbm,
               idx_v, a_v, b_v, sem_a, sem_b):
    wid = lax.axis_index("s") * NC + lax.axis_index("c")
    base = wid * TPW
    pltpu.sync_copy(teamt_hbm.at[pl.ds(base, TPW)], idx_v)
    ca = pltpu.async_copy(mu_hbm.at[idx_v], a_v, sem_a)
    cb = pltpu.async_copy(rho_hbm.at[idx_v], b_v, sem_b)
    ca.wait()
    cb.wait()
    pltpu.sync_copy(a_v, mug_hbm.at[pl.ds(base, TPW)])
    pltpu.sync_copy(b_v, rhog_hbm.at[pl.ds(base, TPW)])


def _tf_bits(c2):
    ks0, ks1 = _K1, _K2
    ks2 = np.uint32(ks0 ^ ks1 ^ np.uint32(0x1BD11BDA))
    x0 = jnp.full_like(c2, ks0)
    x1 = c2 + ks1
    r_a = (13, 15, 26, 6)
    r_b = (17, 29, 16, 24)
    sched = [(ks1, ks2, 1), (ks2, ks0, 2), (ks0, ks1, 3),
             (ks1, ks2, 4), (ks2, ks0, 5)]
    rots = [r_a, r_b, r_a, r_b, r_a]
    for (a, b, inc), rs in zip(sched, rots):
        for r in rs:
            x0 = x0 + x1
            x1 = (x1 << np.uint32(r)) | (x1 >> np.uint32(32 - r))
            x1 = x0 ^ x1
        x0 = x0 + a
        x1 = x1 + np.uint32(b + np.uint32(inc))
    return x0 ^ x1


_ERF_SMALL = [np.float32(v) for v in (
    2.81022636e-08, 3.43273939e-07, -3.5233877e-06,
    -4.39150654e-06, 0.00021858087, -0.00125372503,
    -0.00417768164, 0.246640727, 1.50140941)]
_ERF_LARGE = [np.float32(v) for v in (
    -0.000200214257, 0.000100950558, 0.00134934322,
    -0.00367342844, 0.00573950773, -0.0076224613,
    0.00943887047, 1.00167406, 2.83297682)]


def _erfinv(x):
    w = -jnp.log1p(-x * x)
    small = w < np.float32(5.0)
    ws = w - np.float32(2.5)
    wl = jnp.sqrt(w) - np.float32(3.0)
    ps = _ERF_SMALL[0]
    for c in _ERF_SMALL[1:]:
        ps = ps * ws + c
    pL = _ERF_LARGE[0]
    for c in _ERF_LARGE[1:]:
        pL = pL * wl + c
    return jnp.where(small, ps, pL) * x


_U_LO = np.nextafter(np.float32(-1.0), np.float32(0.0), dtype=np.float32)
_U_SCALE = np.float32(np.float32(1.0) - _U_LO)
_SQRT2 = np.float32(np.sqrt(2.0))


def _eps_at(c2):
    bits = _tf_bits(c2)
    fb = (bits >> np.uint32(9)) | np.uint32(0x3F800000)
    f = lax.bitcast_convert_type(fb, jnp.float32) - np.float32(1.0)
    u = lax.max(jnp.asarray(_U_LO), f * _U_SCALE + _U_LO)
    return _SQRT2 * _erfinv(u)


TCB = 1024
NBLK = B // TCB
RPB = T * TCB // 128
ORB = S * 8


def _perf_tc_body(one_ref, teamt_ref, mug_ref, rhog_ref, out_ref):
    one = one_ref[0]
    z = jnp.zeros((8, 128), jnp.float32)

    def step(t, accs):
        r = pl.multiple_of(t * 8, 8)
        p = teamt_ref[pl.ds(r, 8), :].astype(jnp.uint32)
        mug = mug_ref[pl.ds(r, 8), :]
        sig = jnp.log1p(jnp.exp(rhog_ref[pl.ds(r, 8), :]))
        new = [accs[s] + _eps_at(p + np.uint32(s * N_PLAYER)) * sig
               for s in range(S)]
        return tuple(new) + (accs[S] + mug,)

    accs = lax.fori_loop(0, T, step, (z,) * (S + 1))
    for s in range(S):
        out_ref[pl.ds(s * 8, 8), :] = (accs[s] + accs[S]) * one


def _perf_tc(one, teamt2, mug2, rhog2):
    return pl.pallas_call(
        _perf_tc_body,
        grid=(NBLK,),
        in_specs=[
            pl.BlockSpec(memory_space=pltpu.SMEM),
            pl.BlockSpec((RPB, 128), lambda i: (i, 0)),
            pl.BlockSpec((RPB, 128), lambda i: (i, 0)),
            pl.BlockSpec((RPB, 128), lambda i: (i, 0)),
        ],
        out_specs=pl.BlockSpec((ORB, 128), lambda i: (i, 0)),
        out_shape=jax.ShapeDtypeStruct((NBLK * ORB, 128), jnp.float32),
        compiler_params=pltpu.CompilerParams(skip_device_barrier=True),
    )(one, teamt2, mug2, rhog2)


def kernel(team, num_samples, mu, rho):
    one = ((jnp.asarray(num_samples) - (S - 1)).astype(jnp.float32)
           .reshape(1))
    teamt = team.reshape(NBLK, TCB, T).transpose(0, 2, 1).reshape(B * T)
    mug, rhog = _gather_sc(teamt, mu, rho)
    out = _perf_tc(one, teamt.reshape(B * T // 128, 128),
                   mug.reshape(B * T // 128, 128),
                   rhog.reshape(B * T // 128, 128))
    return (out.reshape(NBLK, S, TCB)
            .transpose(1, 0, 2).reshape(S, B))


# speedup vs baseline: 1.1056x; 1.1056x over previous
import functools

import jax
import jax.numpy as jnp
import numpy as np
from jax import lax
from jax.experimental import pallas as pl
from jax.experimental.pallas import tpu as pltpu
from jax.experimental.pallas import tpu_sc as plsc

N_PLAYER = 1000000
S = 8
B = 16384
T = 20

NC = 2
NS = 16
NW = NC * NS
BH = B
TPW = (BH * T) // NW

_K1 = np.uint32(0)
_K2 = np.uint32(42)

_mesh = plsc.VectorSubcoreMesh(core_axis_name="c", subcore_axis_name="s")


@functools.partial(
    pl.kernel,
    out_type=(
        jax.ShapeDtypeStruct((BH * T,), jnp.float32),
        jax.ShapeDtypeStruct((BH * T,), jnp.float32),
    ),
    mesh=_mesh,
    scratch_types=[
        pltpu.VMEM((TPW,), jnp.int32),
        pltpu.VMEM((TPW,), jnp.float32),
        pltpu.VMEM((TPW,), jnp.float32),
        pltpu.SemaphoreType.DMA,
        pltpu.SemaphoreType.DMA,
    ],
    compiler_params=pltpu.CompilerParams(use_tc_tiling_on_sc=False),
)
def _gather_sc(teamt_hbm, mu_hbm, rho_hbm, mug_hbm, rhog_hbm,
               idx_v, a_v, b_v, sem_a, sem_b):
    wid = lax.axis_index("s") * NC + lax.axis_index("c")
    base = wid * TPW
    pltpu.sync_copy(teamt_hbm.at[pl.ds(base, TPW)], idx_v)
    ca = pltpu.async_copy(mu_hbm.at[idx_v], a_v, sem_a)
    cb = pltpu.async_copy(rho_hbm.at[idx_v], b_v, sem_b)
    ca.wait()
    cb.wait()
    pltpu.sync_copy(a_v, mug_hbm.at[pl.ds(base, TPW)])
    pltpu.sync_copy(b_v, rhog_hbm.at[pl.ds(base, TPW)])


def _tf_bits(c2):
    ks0, ks1 = _K1, _K2
    ks2 = np.uint32(ks0 ^ ks1 ^ np.uint32(0x1BD11BDA))
    x0 = jnp.full_like(c2, ks0)
    x1 = c2 + ks1
    r_a = (13, 15, 26, 6)
    r_b = (17, 29, 16, 24)
    sched = [(ks1, ks2, 1), (ks2, ks0, 2), (ks0, ks1, 3),
             (ks1, ks2, 4), (ks2, ks0, 5)]
    rots = [r_a, r_b, r_a, r_b, r_a]
    for (a, b, inc), rs in zip(sched, rots):
        for r in rs:
            x0 = x0 + x1
            x1 = (x1 << np.uint32(r)) | (x1 >> np.uint32(32 - r))
            x1 = x0 ^ x1
        x0 = x0 + a
        x1 = x1 + np.uint32(b + np.uint32(inc))
    return x0 ^ x1


_ERF_SMALL = [np.float32(v) for v in (
    2.81022636e-08, 3.43273939e-07, -3.5233877e-06,
    -4.39150654e-06, 0.00021858087, -0.00125372503,
    -0.00417768164, 0.246640727, 1.50140941)]
_ERF_LARGE = [np.float32(v) for v in (
    -0.000200214257, 0.000100950558, 0.00134934322,
    -0.00367342844, 0.00573950773, -0.0076224613,
    0.00943887047, 1.00167406, 2.83297682)]


def _erfinv(x):
    w = -jnp.log1p(-x * x)
    small = w < np.float32(5.0)
    ws = w - np.float32(2.5)
    wl = jnp.sqrt(w) - np.float32(3.0)
    ps = _ERF_SMALL[0]
    for c in _ERF_SMALL[1:]:
        ps = ps * ws + c
    pL = _ERF_LARGE[0]
    for c in _ERF_LARGE[1:]:
        pL = pL * wl + c
    return jnp.where(small, ps, pL) * x


_U_LO = np.nextafter(np.float32(-1.0), np.float32(0.0), dtype=np.float32)
_U_SCALE = np.float32(np.float32(1.0) - _U_LO)
_SQRT2 = np.float32(np.sqrt(2.0))


def _eps_at(c2):
    bits = _tf_bits(c2)
    fb = (bits >> np.uint32(9)) | np.uint32(0x3F800000)
    f = lax.bitcast_convert_type(fb, jnp.float32) - np.float32(1.0)
    u = lax.max(jnp.asarray(_U_LO), f * _U_SCALE + _U_LO)
    return _SQRT2 * _erfinv(u)


TCB = 1024
NBLK = B // TCB
RPB = T * TCB // 128
ORB = S * 8


def _perf_tc_body(one_ref, teamt_ref, mug_ref, rhog_ref, out_ref):
    one = one_ref[0]
    z = jnp.zeros((8, 128), jnp.float32)

    def step(t, accs):
        new = list(accs)
        for k in range(2):
            r = pl.multiple_of(t * 16 + k * 8, 8)
            p = teamt_ref[pl.ds(r, 8), :].astype(jnp.uint32)
            mug = mug_ref[pl.ds(r, 8), :]
            sig = jnp.log1p(jnp.exp(rhog_ref[pl.ds(r, 8), :]))
            for s in range(S):
                new[s] = new[s] + _eps_at(p + np.uint32(s * N_PLAYER)) * sig
            new[S] = new[S] + mug
        return tuple(new)

    accs = lax.fori_loop(0, T // 2, step, (z,) * (S + 1))
    for s in range(S):
        out_ref[pl.ds(s * 8, 8), :] = (accs[s] + accs[S]) * one


def _perf_tc(one, teamt2, mug2, rhog2):
    return pl.pallas_call(
        _perf_tc_body,
        grid=(NBLK,),
        in_specs=[
            pl.BlockSpec(memory_space=pltpu.SMEM),
            pl.BlockSpec((RPB, 128), lambda i: (i, 0)),
            pl.BlockSpec((RPB, 128), lambda i: (i, 0)),
            pl.BlockSpec((RPB, 128), lambda i: (i, 0)),
        ],
        out_specs=pl.BlockSpec((ORB, 128), lambda i: (i, 0)),
        out_shape=jax.ShapeDtypeStruct((NBLK * ORB, 128), jnp.float32),
        compiler_params=pltpu.CompilerParams(skip_device_barrier=True),
    )(one, teamt2, mug2, rhog2)


def kernel(team, num_samples, mu, rho):
    one = ((jnp.asarray(num_samples) - (S - 1)).astype(jnp.float32)
           .reshape(1))
    teamt = team.reshape(NBLK, TCB, T).transpose(0, 2, 1).reshape(B * T)
    mug, rhog = _gather_sc(teamt, mu, rho)
    out = _perf_tc(one, teamt.reshape(B * T // 128, 128),
                   mug.reshape(B * T // 128, 128),
                   rhog.reshape(B * T // 128, 128))
    return (out.reshape(NBLK, S, TCB)
            .transpose(1, 0, 2).reshape(S, B))


# speedup vs baseline: 1.1422x; 1.0331x over previous
import functools

import jax
import jax.numpy as jnp
import numpy as np
from jax import lax
from jax.experimental import pallas as pl
from jax.experimental.pallas import tpu as pltpu
from jax.experimental.pallas import tpu_sc as plsc

N_PLAYER = 1000000
S = 8
B = 16384
T = 20

NC = 2
NS = 16
NW = NC * NS
BH = B
TPW = (BH * T) // NW

_K1 = np.uint32(0)
_K2 = np.uint32(42)

_mesh = plsc.VectorSubcoreMesh(core_axis_name="c", subcore_axis_name="s")


@functools.partial(
    pl.kernel,
    out_type=(
        jax.ShapeDtypeStruct((BH * T,), jnp.float32),
        jax.ShapeDtypeStruct((BH * T,), jnp.float32),
    ),
    mesh=_mesh,
    scratch_types=[
        pltpu.VMEM((TPW,), jnp.int32),
        pltpu.VMEM((TPW,), jnp.float32),
        pltpu.VMEM((TPW,), jnp.float32),
        pltpu.SemaphoreType.DMA,
        pltpu.SemaphoreType.DMA,
    ],
    compiler_params=pltpu.CompilerParams(use_tc_tiling_on_sc=False),
)
def _gather_sc(teamt_hbm, mu_hbm, rho_hbm, mug_hbm, rhog_hbm,
               idx_v, a_v, b_v, sem_a, sem_b):
    wid = lax.axis_index("s") * NC + lax.axis_index("c")
    base = wid * TPW
    pltpu.sync_copy(teamt_hbm.at[pl.ds(base, TPW)], idx_v)
    ca = pltpu.async_copy(mu_hbm.at[idx_v], a_v, sem_a)
    cb = pltpu.async_copy(rho_hbm.at[idx_v], b_v, sem_b)
    ca.wait()
    cb.wait()
    pltpu.sync_copy(a_v, mug_hbm.at[pl.ds(base, TPW)])
    pltpu.sync_copy(b_v, rhog_hbm.at[pl.ds(base, TPW)])


def _tf_bits(c2):
    ks0, ks1 = _K1, _K2
    ks2 = np.uint32(ks0 ^ ks1 ^ np.uint32(0x1BD11BDA))
    x0 = jnp.full_like(c2, ks0)
    x1 = c2 + ks1
    r_a = (13, 15, 26, 6)
    r_b = (17, 29, 16, 24)
    sched = [(ks1, ks2, 1), (ks2, ks0, 2), (ks0, ks1, 3),
             (ks1, ks2, 4), (ks2, ks0, 5)]
    rots = [r_a, r_b, r_a, r_b, r_a]
    for (a, b, inc), rs in zip(sched, rots):
        for r in rs:
            x0 = x0 + x1
            x1 = (x1 << np.uint32(r)) | (x1 >> np.uint32(32 - r))
            x1 = x0 ^ x1
        x0 = x0 + a
        x1 = x1 + np.uint32(b + np.uint32(inc))
    return x0 ^ x1


_ERF_SMALL = [np.float32(v) for v in (
    2.81022636e-08, 3.43273939e-07, -3.5233877e-06,
    -4.39150654e-06, 0.00021858087, -0.00125372503,
    -0.00417768164, 0.246640727, 1.50140941)]
_ERF_LARGE = [np.float32(v) for v in (
    -0.000200214257, 0.000100950558, 0.00134934322,
    -0.00367342844, 0.00573950773, -0.0076224613,
    0.00943887047, 1.00167406, 2.83297682)]


def _erfinv(x):
    w = -jnp.log1p(-x * x)
    small = w < np.float32(5.0)
    ws = w - np.float32(2.5)
    wl = jnp.sqrt(w) - np.float32(3.0)
    ps = _ERF_SMALL[0]
    for c in _ERF_SMALL[1:]:
        ps = ps * ws + c
    pL = _ERF_LARGE[0]
    for c in _ERF_LARGE[1:]:
        pL = pL * wl + c
    return jnp.where(small, ps, pL) * x


_U_LO = np.nextafter(np.float32(-1.0), np.float32(0.0), dtype=np.float32)
_U_SCALE = np.float32(np.float32(1.0) - _U_LO)
_SQRT2 = np.float32(np.sqrt(2.0))


def _eps_at(c2):
    bits = _tf_bits(c2)
    fb = (bits >> np.uint32(9)) | np.uint32(0x3F800000)
    f = lax.bitcast_convert_type(fb, jnp.float32) - np.float32(1.0)
    u = lax.max(jnp.asarray(_U_LO), f * _U_SCALE + _U_LO)
    return _SQRT2 * _erfinv(u)


TCB = 1024
NBLK = B // TCB
RPB = T * TCB // 128
ORB = S * 8


def _perf_tc_body(one_ref, teamt_ref, mug_ref, rhog_ref, out_ref):
    one = one_ref[0]
    z = jnp.zeros((8, 128), jnp.float32)

    def step(t, accs):
        new = list(accs)
        for k in range(4):
            r = pl.multiple_of(t * 32 + k * 8, 8)
            p = teamt_ref[pl.ds(r, 8), :].astype(jnp.uint32)
            mug = mug_ref[pl.ds(r, 8), :]
            sig = jnp.log1p(jnp.exp(rhog_ref[pl.ds(r, 8), :]))
            for s in range(S):
                new[s] = new[s] + _eps_at(p + np.uint32(s * N_PLAYER)) * sig
            new[S] = new[S] + mug
        return tuple(new)

    accs = lax.fori_loop(0, T // 4, step, (z,) * (S + 1))
    for s in range(S):
        out_ref[pl.ds(s * 8, 8), :] = (accs[s] + accs[S]) * one


def _perf_tc(one, teamt2, mug2, rhog2):
    return pl.pallas_call(
        _perf_tc_body,
        grid=(NBLK,),
        in_specs=[
            pl.BlockSpec(memory_space=pltpu.SMEM),
            pl.BlockSpec((RPB, 128), lambda i: (i, 0)),
            pl.BlockSpec((RPB, 128), lambda i: (i, 0)),
            pl.BlockSpec((RPB, 128), lambda i: (i, 0)),
        ],
        out_specs=pl.BlockSpec((ORB, 128), lambda i: (i, 0)),
        out_shape=jax.ShapeDtypeStruct((NBLK * ORB, 128), jnp.float32),
        compiler_params=pltpu.CompilerParams(skip_device_barrier=True),
    )(one, teamt2, mug2, rhog2)


def kernel(team, num_samples, mu, rho):
    one = ((jnp.asarray(num_samples) - (S - 1)).astype(jnp.float32)
           .reshape(1))
    teamt = team.reshape(NBLK, TCB, T).transpose(0, 2, 1).reshape(B * T)
    mug, rhog = _gather_sc(teamt, mu, rho)
    out = _perf_tc(one, teamt.reshape(B * T // 128, 128),
                   mug.reshape(B * T // 128, 128),
                   rhog.reshape(B * T // 128, 128))
    return (out.reshape(NBLK, S, TCB)
            .transpose(1, 0, 2).reshape(S, B))


# speedup vs baseline: 1.1603x; 1.0159x over previous
import functools

import jax
import jax.numpy as jnp
import numpy as np
from jax import lax
from jax.experimental import pallas as pl
from jax.experimental.pallas import tpu as pltpu
from jax.experimental.pallas import tpu_sc as plsc

N_PLAYER = 1000000
S = 8
B = 16384
T = 20

NC = 2
NS = 16
NW = NC * NS
BH = B
TPW = (BH * T) // NW

_K1 = np.uint32(0)
_K2 = np.uint32(42)

_mesh = plsc.VectorSubcoreMesh(core_axis_name="c", subcore_axis_name="s")


@functools.partial(
    pl.kernel,
    out_type=(
        jax.ShapeDtypeStruct((BH * T,), jnp.float32),
        jax.ShapeDtypeStruct((BH * T,), jnp.float32),
    ),
    mesh=_mesh,
    scratch_types=[
        pltpu.VMEM((TPW,), jnp.int32),
        pltpu.VMEM((TPW,), jnp.float32),
        pltpu.VMEM((TPW,), jnp.float32),
        pltpu.SemaphoreType.DMA,
        pltpu.SemaphoreType.DMA,
    ],
    compiler_params=pltpu.CompilerParams(use_tc_tiling_on_sc=False),
)
def _gather_sc(teamt_hbm, mu_hbm, rho_hbm, mug_hbm, rhog_hbm,
               idx_v, a_v, b_v, sem_a, sem_b):
    wid = lax.axis_index("s") * NC + lax.axis_index("c")
    base = wid * TPW
    pltpu.sync_copy(teamt_hbm.at[pl.ds(base, TPW)], idx_v)
    ca = pltpu.async_copy(mu_hbm.at[idx_v], a_v, sem_a)
    cb = pltpu.async_copy(rho_hbm.at[idx_v], b_v, sem_b)
    ca.wait()
    cb.wait()
    pltpu.sync_copy(a_v, mug_hbm.at[pl.ds(base, TPW)])
    pltpu.sync_copy(b_v, rhog_hbm.at[pl.ds(base, TPW)])


def _tf_bits(c2):
    ks0, ks1 = _K1, _K2
    ks2 = np.uint32(ks0 ^ ks1 ^ np.uint32(0x1BD11BDA))
    x0 = jnp.full_like(c2, ks0)
    x1 = c2 + ks1
    r_a = (13, 15, 26, 6)
    r_b = (17, 29, 16, 24)
    sched = [(ks1, ks2, 1), (ks2, ks0, 2), (ks0, ks1, 3),
             (ks1, ks2, 4), (ks2, ks0, 5)]
    rots = [r_a, r_b, r_a, r_b, r_a]
    for (a, b, inc), rs in zip(sched, rots):
        for r in rs:
            x0 = x0 + x1
            x1 = (x1 << np.uint32(r)) | (x1 >> np.uint32(32 - r))
            x1 = x0 ^ x1
        x0 = x0 + a
        x1 = x1 + np.uint32(b + np.uint32(inc))
    return x0 ^ x1


_ERF_SMALL = [np.float32(v) for v in (
    2.81022636e-08, 3.43273939e-07, -3.5233877e-06,
    -4.39150654e-06, 0.00021858087, -0.00125372503,
    -0.00417768164, 0.246640727, 1.50140941)]
_ERF_LARGE = [np.float32(v) for v in (
    -0.000200214257, 0.000100950558, 0.00134934322,
    -0.00367342844, 0.00573950773, -0.0076224613,
    0.00943887047, 1.00167406, 2.83297682)]


def _erfinv(x):
    w = -jnp.log1p(-x * x)
    small = w < np.float32(5.0)
    ws = w - np.float32(2.5)
    wl = jnp.sqrt(w) - np.float32(3.0)
    ps = _ERF_SMALL[0]
    for c in _ERF_SMALL[1:]:
        ps = ps * ws + c
    pL = _ERF_LARGE[0]
    for c in _ERF_LARGE[1:]:
        pL = pL * wl + c
    return jnp.where(small, ps, pL) * x


_U_LO = np.nextafter(np.float32(-1.0), np.float32(0.0), dtype=np.float32)
_U_SCALE = np.float32(np.float32(1.0) - _U_LO)
_SQRT2 = np.float32(np.sqrt(2.0))


def _eps_at(c2):
    bits = _tf_bits(c2)
    fb = (bits >> np.uint32(9)) | np.uint32(0x3F800000)
    f = lax.bitcast_convert_type(fb, jnp.float32) - np.float32(1.0)
    u = lax.max(jnp.asarray(_U_LO), f * _U_SCALE + _U_LO)
    return _SQRT2 * _erfinv(u)


TCB = 1024
NBLK = B // TCB
RPB = T * TCB // 128
ORB = S * 8


def _perf_tc_body(one_ref, teamt_ref, mug_ref, rhog_ref, out_ref):
    one = one_ref[0]
    z = jnp.zeros((8, 128), jnp.float32)

    accs = [z] * (S + 1)
    for t in range(T):
        p = teamt_ref[pl.ds(t * 8, 8), :].astype(jnp.uint32)
        mug = mug_ref[pl.ds(t * 8, 8), :]
        sig = jnp.log1p(jnp.exp(rhog_ref[pl.ds(t * 8, 8), :]))
        for s in range(S):
            accs[s] = accs[s] + _eps_at(p + np.uint32(s * N_PLAYER)) * sig
        accs[S] = accs[S] + mug
    for s in range(S):
        out_ref[pl.ds(s * 8, 8), :] = (accs[s] + accs[S]) * one


def _perf_tc(one, teamt2, mug2, rhog2):
    return pl.pallas_call(
        _perf_tc_body,
        grid=(NBLK,),
        in_specs=[
            pl.BlockSpec(memory_space=pltpu.SMEM),
            pl.BlockSpec((RPB, 128), lambda i: (i, 0)),
            pl.BlockSpec((RPB, 128), lambda i: (i, 0)),
            pl.BlockSpec((RPB, 128), lambda i: (i, 0)),
        ],
        out_specs=pl.BlockSpec((ORB, 128), lambda i: (i, 0)),
        out_shape=jax.ShapeDtypeStruct((NBLK * ORB, 128), jnp.float32),
        compiler_params=pltpu.CompilerParams(skip_device_barrier=True),
    )(one, teamt2, mug2, rhog2)


def kernel(team, num_samples, mu, rho):
    one = ((jnp.asarray(num_samples) - (S - 1)).astype(jnp.float32)
           .reshape(1))
    teamt = team.reshape(NBLK, TCB, T).transpose(0, 2, 1).reshape(B * T)
    mug, rhog = _gather_sc(teamt, mu, rho)
    out = _perf_tc(one, teamt.reshape(B * T // 128, 128),
                   mug.reshape(B * T // 128, 128),
                   rhog.reshape(B * T // 128, 128))
    return (out.reshape(NBLK, S, TCB)
            .transpose(1, 0, 2).reshape(S, B))


# speedup vs baseline: 1.1784x; 1.0156x over previous
import functools

import jax
import jax.numpy as jnp
import numpy as np
from jax import lax
from jax.experimental import pallas as pl
from jax.experimental.pallas import tpu as pltpu
from jax.experimental.pallas import tpu_sc as plsc

N_PLAYER = 1000000
S = 8
B = 16384
T = 20

NC = 2
NS = 16
NW = NC * NS
BH = B
TPW = (BH * T) // NW

_K1 = np.uint32(0)
_K2 = np.uint32(42)

_mesh = plsc.VectorSubcoreMesh(core_axis_name="c", subcore_axis_name="s")


@functools.partial(
    pl.kernel,
    out_type=(
        jax.ShapeDtypeStruct((BH * T,), jnp.float32),
        jax.ShapeDtypeStruct((BH * T,), jnp.float32),
    ),
    mesh=_mesh,
    scratch_types=[
        pltpu.VMEM((TPW,), jnp.int32),
        pltpu.VMEM((TPW,), jnp.float32),
        pltpu.VMEM((TPW,), jnp.float32),
        pltpu.SemaphoreType.DMA,
        pltpu.SemaphoreType.DMA,
    ],
    compiler_params=pltpu.CompilerParams(use_tc_tiling_on_sc=False),
)
def _gather_sc(teamt_hbm, mu_hbm, rho_hbm, mug_hbm, rhog_hbm,
               idx_v, a_v, b_v, sem_a, sem_b):
    wid = lax.axis_index("s") * NC + lax.axis_index("c")
    base = wid * TPW
    pltpu.sync_copy(teamt_hbm.at[pl.ds(base, TPW)], idx_v)
    ca = pltpu.async_copy(mu_hbm.at[idx_v], a_v, sem_a)
    cb = pltpu.async_copy(rho_hbm.at[idx_v], b_v, sem_b)
    ca.wait()
    cb.wait()
    pltpu.sync_copy(a_v, mug_hbm.at[pl.ds(base, TPW)])
    pltpu.sync_copy(b_v, rhog_hbm.at[pl.ds(base, TPW)])


def _tf_bits(c2):
    ks0, ks1 = _K1, _K2
    ks2 = np.uint32(ks0 ^ ks1 ^ np.uint32(0x1BD11BDA))
    x0 = jnp.full_like(c2, ks0)
    x1 = c2 + ks1
    r_a = (13, 15, 26, 6)
    r_b = (17, 29, 16, 24)
    sched = [(ks1, ks2, 1), (ks2, ks0, 2), (ks0, ks1, 3),
             (ks1, ks2, 4), (ks2, ks0, 5)]
    rots = [r_a, r_b, r_a, r_b, r_a]
    for (a, b, inc), rs in zip(sched, rots):
        for r in rs:
            x0 = x0 + x1
            x1 = (x1 << np.uint32(r)) | (x1 >> np.uint32(32 - r))
            x1 = x0 ^ x1
        x0 = x0 + a
        x1 = x1 + np.uint32(b + np.uint32(inc))
    return x0 ^ x1


_ERF_SMALL = [np.float32(v) for v in (
    2.81022636e-08, 3.43273939e-07, -3.5233877e-06,
    -4.39150654e-06, 0.00021858087, -0.00125372503,
    -0.00417768164, 0.246640727, 1.50140941)]
_ERF_LARGE = [np.float32(v) for v in (
    -0.000200214257, 0.000100950558, 0.00134934322,
    -0.00367342844, 0.00573950773, -0.0076224613,
    0.00943887047, 1.00167406, 2.83297682)]


def _erfinv(x):
    w = -jnp.log1p(-x * x)
    small = w < np.float32(5.0)
    ws = w - np.float32(2.5)
    wl = jnp.sqrt(w) - np.float32(3.0)
    ps = _ERF_SMALL[0]
    for c in _ERF_SMALL[1:]:
        ps = ps * ws + c
    pL = _ERF_LARGE[0]
    for c in _ERF_LARGE[1:]:
        pL = pL * wl + c
    return jnp.where(small, ps, pL) * x


_U_LO = np.nextafter(np.float32(-1.0), np.float32(0.0), dtype=np.float32)
_U_SCALE = np.float32(np.float32(1.0) - _U_LO)
_SQRT2 = np.float32(np.sqrt(2.0))


def _eps_at(c2):
    bits = _tf_bits(c2)
    fb = (bits >> np.uint32(9)) | np.uint32(0x3F800000)
    f = lax.bitcast_convert_type(fb, jnp.float32) - np.float32(1.0)
    u = lax.max(jnp.asarray(_U_LO), f * _U_SCALE + _U_LO)
    return _SQRT2 * _erfinv(u)


TCB = 1024
NBLK = B // TCB
RPB = T * TCB // 128


def _perf_tc_body(one_ref, teamt_ref, mug_ref, rhog_ref, out_ref):
    one = one_ref[0]
    z = jnp.zeros((8, 128), jnp.float32)

    accs = [z] * (S + 1)
    for t in range(T):
        p = teamt_ref[pl.ds(t * 8, 8), :].astype(jnp.uint32)
        mug = mug_ref[pl.ds(t * 8, 8), :]
        sig = jnp.log1p(jnp.exp(rhog_ref[pl.ds(t * 8, 8), :]))
        for s in range(S):
            accs[s] = accs[s] + _eps_at(p + np.uint32(s * N_PLAYER)) * sig
        accs[S] = accs[S] + mug
    for s in range(S):
        row = ((accs[s] + accs[S]) * one).reshape(1, TCB)
        out_ref[pl.ds(s, 1), :] = row


def _perf_tc(one, teamt2, mug2, rhog2):
    return pl.pallas_call(
        _perf_tc_body,
        grid=(NBLK,),
        in_specs=[
            pl.BlockSpec(memory_space=pltpu.SMEM),
            pl.BlockSpec((RPB, 128), lambda i: (i, 0)),
            pl.BlockSpec((RPB, 128), lambda i: (i, 0)),
            pl.BlockSpec((RPB, 128), lambda i: (i, 0)),
        ],
        out_specs=pl.BlockSpec((S, TCB), lambda i: (0, i)),
        out_shape=jax.ShapeDtypeStruct((S, B), jnp.float32),
    )(one, teamt2, mug2, rhog2)


def kernel(team, num_samples, mu, rho):
    one = ((jnp.asarray(num_samples) - (S - 1)).astype(jnp.float32)
           .reshape(1))
    teamt = team.reshape(NBLK, TCB, T).transpose(0, 2, 1).reshape(B * T)
    mug, rhog = _gather_sc(teamt, mu, rho)
    return _perf_tc(one, teamt.reshape(B * T // 128, 128),
                    mug.reshape(B * T // 128, 128),
                    rhog.reshape(B * T // 128, 128))
